# baseline (speedup 1.0000x reference)
import functools

import jax
import jax.numpy as jnp
from jax import lax
from jax.experimental import pallas as pl
from jax.experimental.pallas import tpu as pltpu
from jax.experimental.pallas import tpu_sc as plsc

DIM = 64
EPS = 1e-05

NUM_CORES = 2
NUM_SUBCORES = 16
NW = NUM_CORES * NUM_SUBCORES

CHUNK = 128


def _make_gather(batch: int):
  assert batch % (NW * CHUNK) == 0
  b_per_w = batch // NW
  n_chunks = b_per_w // CHUNK
  mesh = plsc.VectorSubcoreMesh(
      core_axis_name="c", subcore_axis_name="s",
      num_cores=NUM_CORES, num_subcores=NUM_SUBCORES)

  @functools.partial(
      pl.kernel,
      out_type=jax.ShapeDtypeStruct((batch, DIM), jnp.float32),
      mesh=mesh,
      compiler_params=pltpu.CompilerParams(use_tc_tiling_on_sc=False),
      scratch_types=[
          pltpu.VMEM((b_per_w,), jnp.int32),
          pltpu.VMEM((CHUNK, DIM), jnp.float32),
          pltpu.SemaphoreType.DMA,
      ],
  )
  def gather_kernel(idx_hbm, table_hbm, out_hbm, idx_v, rows_v, sem):
    wid = lax.axis_index("s") * NUM_CORES + lax.axis_index("c")
    base = wid * b_per_w
    pltpu.sync_copy(idx_hbm.at[pl.ds(base, b_per_w)], idx_v)

    @pl.loop(0, n_chunks)
    def _chunk(c):
      off = c * CHUNK
      pltpu.async_copy(
          table_hbm.at[idx_v.at[pl.ds(off, CHUNK)]], rows_v, sem).wait()
      pltpu.sync_copy(rows_v, out_hbm.at[pl.ds(base + off, CHUNK)])

  return gather_kernel


def _ln_body(x_ref, s_ref, b_ref, o_ref):
  x = x_ref[...]
  mean = jnp.mean(x, axis=-1, keepdims=True)
  var = jnp.mean(jnp.square(x - mean), axis=-1, keepdims=True)
  inv = s_ref[...] * lax.rsqrt(var + EPS)
  o_ref[...] = x * inv + (b_ref[...] - mean * inv)


def _make_ln(batch: int, block: int):
  assert batch % block == 0
  return pl.pallas_call(
      _ln_body,
      grid=(batch // block,),
      in_specs=[
          pl.BlockSpec((block, DIM), lambda i: (i, 0)),
          pl.BlockSpec((1, DIM), lambda i: (0, 0)),
          pl.BlockSpec((1, DIM), lambda i: (0, 0)),
      ],
      out_specs=pl.BlockSpec((block, DIM), lambda i: (i, 0)),
      out_shape=jax.ShapeDtypeStruct((batch, DIM), jnp.float32),
  )


def kernel(inputs, emb_weight, ln_scale, ln_bias):
  idx = jnp.asarray(inputs, jnp.int32).reshape(-1)
  batch = idx.shape[0]
  rows = _make_gather(batch)(idx, emb_weight)
  out = _make_ln(batch, 4096)(
      rows, ln_scale.reshape(1, DIM), ln_bias.reshape(1, DIM))
  return out.reshape(inputs.shape + (DIM,))


# speedup vs baseline: 1.0609x; 1.0609x over previous
import functools

import jax
import jax.numpy as jnp
from jax import lax
from jax.experimental import pallas as pl
from jax.experimental.pallas import tpu as pltpu
from jax.experimental.pallas import tpu_sc as plsc

DIM = 64
EPS = 1e-05

NUM_CORES = 2
NUM_SUBCORES = 16
NW = NUM_CORES * NUM_SUBCORES

CHUNK = 128


def _make_gather(batch: int, vocab_pairs: int):
  assert batch % (NW * CHUNK) == 0
  b_per_w = batch // NW
  n_chunks = b_per_w // CHUNK
  mesh = plsc.VectorSubcoreMesh(
      core_axis_name="c", subcore_axis_name="s",
      num_cores=NUM_CORES, num_subcores=NUM_SUBCORES)

  @functools.partial(
      pl.kernel,
      out_type=jax.ShapeDtypeStruct((batch, 2 * DIM), jnp.float32),
      mesh=mesh,
      scratch_types=[
          pltpu.VMEM((b_per_w,), jnp.int32),
          pltpu.VMEM((CHUNK, 2 * DIM), jnp.float32),
          pltpu.SemaphoreType.DMA,
      ],
  )
  def gather_kernel(idx_hbm, table_hbm, out_hbm, idx_v, rows_v, sem):
    wid = lax.axis_index("s") * NUM_CORES + lax.axis_index("c")
    base = wid * b_per_w
    pltpu.sync_copy(idx_hbm.at[pl.ds(base, b_per_w)], idx_v)

    @pl.loop(0, n_chunks)
    def _chunk(c):
      off = c * CHUNK
      pltpu.async_copy(
          table_hbm.at[idx_v.at[pl.ds(off, CHUNK)]], rows_v, sem).wait()
      pltpu.sync_copy(rows_v, out_hbm.at[pl.ds(base + off, CHUNK)])

  return gather_kernel


def _ln_body(x_ref, pm_ref, s_ref, b_ref, o_ref):
  g, seq, _ = o_ref.shape
  x = x_ref[...].reshape(g, seq, 2 * DIM)
  pm = pm_ref[...].reshape(g, seq, 1)
  xsel = jnp.where(pm > 0.5, x[:, :, DIM:], x[:, :, :DIM])
  mean = jnp.mean(xsel, axis=-1, keepdims=True)
  var = jnp.mean(jnp.square(xsel - mean), axis=-1, keepdims=True)
  inv = s_ref[...] * lax.rsqrt(var + EPS)
  o_ref[...] = xsel * inv + (b_ref[...] - mean * inv)


def _make_ln(groups: int, seq: int, gblk: int):
  assert groups % gblk == 0
  batch = groups * seq
  rblk = gblk * seq
  return pl.pallas_call(
      _ln_body,
      grid=(groups // gblk,),
      in_specs=[
          pl.BlockSpec((rblk, 2 * DIM), lambda i: (i, 0)),
          pl.BlockSpec((gblk, seq), lambda i: (i, 0)),
          pl.BlockSpec((1, 1, DIM), lambda i: (0, 0, 0)),
          pl.BlockSpec((1, 1, DIM), lambda i: (0, 0, 0)),
      ],
      out_specs=pl.BlockSpec((gblk, seq, DIM), lambda i: (i, 0, 0)),
      out_shape=jax.ShapeDtypeStruct((groups, seq, DIM), jnp.float32),
  )


def kernel(inputs, emb_weight, ln_scale, ln_bias):
  groups, seq = inputs.shape
  idx = jnp.asarray(inputs, jnp.int32)
  pair = (idx >> 1).reshape(-1)
  pmask = (idx & 1).astype(jnp.float32)
  table_pairs = emb_weight.reshape(-1, 2 * DIM)
  rows = _make_gather(groups * seq, table_pairs.shape[0])(pair, table_pairs)
  return _make_ln(groups, seq, 64)(
      rows, pmask, ln_scale.reshape(1, 1, DIM), ln_bias.reshape(1, 1, DIM))
